# Initial kernel scaffold; baseline (speedup 1.0000x reference)
import jax
import jax.numpy as jnp
from jax.experimental import pallas as pl


def kernel(x1, x2, norm_adj, edge_index1, train_fts_idx, vali_test_fts_idx, W_fl, b_fl, W_enc, b_enc, W_proj, b_proj):
    raise NotImplementedError("write your pallas kernel here")


# speedup vs baseline: 2.9341x; 2.9341x over previous
import functools

import jax
import jax.numpy as jnp
from jax import lax
from jax.experimental import pallas as pl
from jax.experimental.pallas import tpu as pltpu
from jax.experimental.pallas import tpu_sc as plsc

N = 10000
D = 128
NTILES = 16
NCORES = 2
CHUNK = 128
ACC_ROWS = 10112
SLAB0 = ACC_ROWS // NTILES
SLABW = 624


def _make_agg(chunks_per_worker: int, with_deg: bool):
    mesh = plsc.VectorSubcoreMesh(core_axis_name="c", subcore_axis_name="s",
                                  num_cores=NCORES, num_subcores=NTILES)
    out_type = [jax.ShapeDtypeStruct((NCORES, N, D), jnp.float32)]
    if with_deg:
        out_type.append(jax.ShapeDtypeStruct((NCORES, N, 16), jnp.float32))
    scratch = [
        pltpu.VMEM_SHARED((ACC_ROWS, D), jnp.float32),
        pltpu.VMEM((CHUNK,), jnp.int32),
        pltpu.VMEM((CHUNK,), jnp.int32),
        pltpu.VMEM((CHUNK, D), jnp.float32),
        pltpu.SemaphoreType.DMA,
    ]
    if with_deg:
        scratch.insert(1, pltpu.VMEM_SHARED((ACC_ROWS, 16), jnp.float32))
        scratch.append(pltpu.VMEM((CHUNK, 16), jnp.float32))

    @functools.partial(pl.kernel, mesh=mesh, out_type=tuple(out_type),
                       scratch_types=tuple(scratch))
    def agg(table, src_p, dst_p, z128, z16, ones16, *rest):
        if with_deg:
            (out_sum, out_deg, acc, dacc, idx_s, idx_d, rows, sem, ones_v) = rest
        else:
            (out_sum, acc, idx_s, idx_d, rows, sem) = rest
        c = lax.axis_index("c")
        s = lax.axis_index("s")
        wid = c * NTILES + s

        pltpu.sync_copy(z128, acc.at[pl.ds(s * SLAB0, SLAB0)])
        if with_deg:
            pltpu.sync_copy(z16, dacc.at[pl.ds(s * SLAB0, SLAB0)])
            pltpu.sync_copy(ones16, ones_v)
        plsc.subcore_barrier()

        base = wid * (chunks_per_worker * CHUNK)

        def chunk_body(j, carry):
            off = base + j * CHUNK
            pltpu.sync_copy(src_p.at[pl.ds(off, CHUNK)], idx_s)
            pltpu.sync_copy(dst_p.at[pl.ds(off, CHUNK)], idx_d)
            pltpu.async_copy(table.at[idx_s], rows, sem).wait()
            pltpu.sync_copy(rows, acc.at[idx_d], add=True)
            if with_deg:
                pltpu.sync_copy(ones_v, dacc.at[idx_d], add=True)
            return carry

        lax.fori_loop(0, chunks_per_worker, chunk_body, 0)
        plsc.subcore_barrier()

        @pl.when(s < NTILES - 1)
        def _():
            pltpu.sync_copy(acc.at[pl.ds(s * SLABW, SLABW)],
                            out_sum.at[c, pl.ds(s * SLABW, SLABW)])
            if with_deg:
                pltpu.sync_copy(dacc.at[pl.ds(s * SLABW, SLABW)],
                                out_deg.at[c, pl.ds(s * SLABW, SLABW)])

        @pl.when(s == NTILES - 1)
        def _():
            tail = N - (NTILES - 1) * SLABW
            off = (NTILES - 1) * SLABW
            pltpu.sync_copy(acc.at[pl.ds(off, tail)],
                            out_sum.at[c, pl.ds(off, tail)])
            if with_deg:
                pltpu.sync_copy(dacc.at[pl.ds(off, tail)],
                                out_deg.at[c, pl.ds(off, tail)])

    return agg


def _stage_a(x1, mask, W_fl, b_fl):
    bm = 2000

    def body(x_ref, m_ref, w_ref, b_ref, o_ref):
        xl = jnp.dot(x_ref[...], w_ref[...],
                     preferred_element_type=jnp.float32) + b_ref[...]
        o_ref[...] = x_ref[...] + m_ref[...] * xl

    return pl.pallas_call(
        body,
        grid=(N // bm,),
        in_specs=[
            pl.BlockSpec((bm, D), lambda i: (i, 0)),
            pl.BlockSpec((bm, 1), lambda i: (i, 0)),
            pl.BlockSpec((D, D), lambda i: (0, 0)),
            pl.BlockSpec((1, D), lambda i: (0, 0)),
        ],
        out_specs=pl.BlockSpec((bm, D), lambda i: (i, 0)),
        out_shape=jax.ShapeDtypeStruct((N, D), jnp.float32),
    )(x1, mask, W_fl, b_fl.reshape(1, D))


def _stage_b(norm_adj, X1p):
    bm = 200

    def body(a_ref, x_ref, o_ref):
        o_ref[...] = jnp.dot(a_ref[...], x_ref[...],
                             preferred_element_type=jnp.float32)

    return pl.pallas_call(
        body,
        grid=(N // bm,),
        in_specs=[
            pl.BlockSpec((bm, N), lambda m: (m, 0)),
            pl.BlockSpec((N, D), lambda m: (0, 0)),
        ],
        out_specs=pl.BlockSpec((bm, D), lambda m: (m, 0)),
        out_shape=jax.ShapeDtypeStruct((N, D), jnp.float32),
        compiler_params=pltpu.CompilerParams(
            dimension_semantics=("arbitrary",)),
    )(norm_adj, X1p)


def _stage_c(x_1, x2, S1p, S2p, DEGp, W_enc, b_enc, W_proj, b_proj):
    bm = 2000

    def body(x1_ref, x2_ref, s1_ref, s2_ref, dg_ref, we_ref, be_ref,
             wp_ref, bp_ref, o_ref):
        deg = dg_ref[0, :, :1] + dg_ref[1, :, :1]
        inv = 1.0 / jnp.clip(deg, 1.0, None)
        agg1 = (s1_ref[0] + s1_ref[1]) * inv
        agg2 = (s2_ref[0] + s2_ref[1]) * inv
        z1 = jnp.maximum(
            jnp.dot(x1_ref[...] + agg1, we_ref[...],
                    preferred_element_type=jnp.float32) + be_ref[...], 0.0)
        z2 = jnp.maximum(
            jnp.dot(x2_ref[...] + agg2, we_ref[...],
                    preferred_element_type=jnp.float32) + be_ref[...], 0.0)
        z = (z1 + z2) * 0.5
        o_ref[...] = jnp.dot(z, wp_ref[...],
                             preferred_element_type=jnp.float32) + bp_ref[...]

    return pl.pallas_call(
        body,
        grid=(N // bm,),
        in_specs=[
            pl.BlockSpec((bm, D), lambda i: (i, 0)),
            pl.BlockSpec((bm, D), lambda i: (i, 0)),
            pl.BlockSpec((NCORES, bm, D), lambda i: (0, i, 0)),
            pl.BlockSpec((NCORES, bm, D), lambda i: (0, i, 0)),
            pl.BlockSpec((NCORES, bm, 16), lambda i: (0, i, 0)),
            pl.BlockSpec((D, D), lambda i: (0, 0)),
            pl.BlockSpec((1, D), lambda i: (0, 0)),
            pl.BlockSpec((D, D), lambda i: (0, 0)),
            pl.BlockSpec((1, D), lambda i: (0, 0)),
        ],
        out_specs=pl.BlockSpec((bm, D), lambda i: (i, 0)),
        out_shape=jax.ShapeDtypeStruct((N, D), jnp.float32),
    )(x_1, x2, S1p, S2p, DEGp, W_enc, b_enc.reshape(1, D),
      W_proj, b_proj.reshape(1, D))


def kernel(x1, x2, norm_adj, edge_index1, train_fts_idx, vali_test_fts_idx,
           W_fl, b_fl, W_enc, b_enc, W_proj, b_proj):
    E = edge_index1.shape[1]
    nworkers = NCORES * NTILES
    cpw = (-(-E // nworkers) + CHUNK - 1) // CHUNK
    e_pad = nworkers * cpw * CHUNK

    src = edge_index1[0].astype(jnp.int32)
    dst = edge_index1[1].astype(jnp.int32)
    pad = e_pad - E
    src_p = jnp.concatenate([src, jnp.zeros((pad,), jnp.int32)])
    dst_p = jnp.concatenate([dst, jnp.full((pad,), N, jnp.int32)])

    mask = jnp.zeros((N, 1), jnp.float32).at[vali_test_fts_idx].set(1.0)

    z128 = jnp.zeros((SLAB0, D), jnp.float32)
    z16 = jnp.zeros((SLAB0, 16), jnp.float32)
    ones16 = jnp.ones((CHUNK, 16), jnp.float32)

    agg_deg = _make_agg(cpw, with_deg=True)
    agg_plain = _make_agg(cpw, with_deg=False)

    (S2p,) = agg_plain(x2, src_p, dst_p, z128, z16, ones16)
    dg = jax.ops.segment_sum(jnp.ones((E,), jnp.float32), dst,
                             num_segments=N)
    DEGp = jnp.stack([jnp.broadcast_to(dg[:, None], (N, 16)),
                      jnp.zeros((N, 16), jnp.float32)])
    X1p = _stage_a(x1, mask, W_fl, b_fl)
    x_1 = _stage_b(norm_adj, X1p)
    (S1p,) = agg_plain(x_1, src_p, dst_p, z128, z16, ones16)

    return _stage_c(x_1, x2, S1p, S2p, DEGp, W_enc, b_enc, W_proj, b_proj)
